# Initial kernel scaffold; baseline (speedup 1.0000x reference)
import jax
import jax.numpy as jnp
from jax.experimental import pallas as pl


def kernel(y_pred, y_true):
    raise NotImplementedError("write your pallas kernel here")


# speedup vs baseline: 23.6192x; 23.6192x over previous
import functools

import jax
import jax.numpy as jnp
from jax import lax
from jax.experimental import pallas as pl
from jax.experimental.pallas import tpu as pltpu
from jax.experimental.pallas import tpu_sc as plsc

N = 2097152
D = 64
BINS = 65
NC = 2
NS = 16
L = 16
NW = NC * NS
RPW = N // NW
R = 256
CH = RPW // R
W = R * D
GPC = R // L
HW = BINS * L

_mesh = plsc.VectorSubcoreMesh(
    core_axis_name="c", subcore_axis_name="s", num_cores=NC, num_subcores=NS
)


@functools.partial(
    pl.kernel,
    out_type=[
        jax.ShapeDtypeStruct((NW, HW), jnp.float32),
        jax.ShapeDtypeStruct((NW, L), jnp.float32),
    ],
    mesh=_mesh,
    compiler_params=pltpu.CompilerParams(needs_layout_passes=False),
    scratch_types=[
        pltpu.VMEM((W,), jnp.float32),
        pltpu.VMEM((W,), jnp.float32),
        pltpu.VMEM((W,), jnp.float32),
        pltpu.VMEM((W,), jnp.float32),
        pltpu.VMEM((HW,), jnp.float32),
        pltpu.VMEM((L,), jnp.float32),
        pltpu.SemaphoreType.DMA,
        pltpu.SemaphoreType.DMA,
        pltpu.SemaphoreType.DMA,
        pltpu.SemaphoreType.DMA,
    ],
)
def _sc_hamming(a_hbm, b_hbm, hist_out, sum_out,
                a0, a1, b0, b1, hist_v, sum_v, sa0, sa1, sb0, sb1):
    wid = lax.axis_index("s") * NC + lax.axis_index("c")
    base = wid * (RPW * D)

    zeros = jnp.zeros((L,), jnp.float32)
    ones = jnp.ones((L,), jnp.float32)
    lane = lax.iota(jnp.int32, L)
    rowsel = lane * D

    for i in range(BINS):
        hist_v[pl.ds(i * L, L)] = zeros
    sum_v[...] = zeros

    def issue(g, aref, bref, sa, sb):
        off = base + g * W
        pltpu.async_copy(a_hbm.at[pl.ds(off, W)], aref, sa)
        pltpu.async_copy(b_hbm.at[pl.ds(off, W)], bref, sb)

    def wait(aref, bref, sa, sb):
        pltpu.make_async_copy(a_hbm.at[pl.ds(0, W)], aref, sa).wait()
        pltpu.make_async_copy(b_hbm.at[pl.ds(0, W)], bref, sb).wait()

    def compute(aref, bref):
        a_t = aref.at[pl.ds(0, W)]
        b_t = bref.at[pl.ds(0, W)]
        hist_t = hist_v.at[pl.ds(0, HW)]
        sum_t = sum_v.at[pl.ds(0, L)]

        def grp(i, carry):
            rbase = i * (L * D) + rowsel
            acc = zeros
            for c in range(D):
                idx = rbase + c
                av = plsc.load_gather(a_t, [idx])
                bv = plsc.load_gather(b_t, [idx])
                acc = acc + jnp.abs(av - bv)
            d = acc.astype(jnp.int32)
            plsc.addupdate_scatter(hist_t, [d * L + lane], ones)
            plsc.addupdate(sum_t, acc)
            return carry
        lax.fori_loop(0, GPC, grp, 0)

    issue(0, a0, b0, sa0, sb0)
    issue(1, a1, b1, sa1, sb1)

    def outer(t, carry):
        g = t * 2
        wait(a0, b0, sa0, sb0)
        compute(a0, b0)

        @pl.when(g + 2 < CH)
        def _():
            issue(g + 2, a0, b0, sa0, sb0)

        wait(a1, b1, sa1, sb1)
        compute(a1, b1)

        @pl.when(g + 3 < CH)
        def _():
            issue(g + 3, a1, b1, sa1, sb1)

        return carry

    lax.fori_loop(0, CH // 2, outer, 0)

    pltpu.sync_copy(hist_v, hist_out.at[wid])
    pltpu.sync_copy(sum_v, sum_out.at[wid])


def kernel(y_pred, y_true):
    hist_parts, sum_parts = _sc_hamming(y_pred.reshape(-1), y_true.reshape(-1))
    histogram = hist_parts.reshape(NW, BINS, L).sum(axis=(0, 2))
    mean = sum_parts.sum() / jnp.float32(N)
    return mean, histogram


# speedup vs baseline: 55.3857x; 2.3449x over previous
import functools

import jax
import jax.numpy as jnp
from jax import lax
from jax.experimental import pallas as pl
from jax.experimental.pallas import tpu as pltpu
from jax.experimental.pallas import tpu_sc as plsc

N = 2097152
D = 64
BINS = 65
NC = 2
NS = 16
L = 16
NW = NC * NS
RPW = N // NW
R = 256
CH = RPW // R
W = R * D
GPC = R // L
HW = BINS * L

_mesh = plsc.VectorSubcoreMesh(
    core_axis_name="c", subcore_axis_name="s", num_cores=NC, num_subcores=NS
)


@functools.partial(
    pl.kernel,
    out_type=[
        jax.ShapeDtypeStruct((NW, HW), jnp.float32),
        jax.ShapeDtypeStruct((NW, L), jnp.float32),
    ],
    mesh=_mesh,
    compiler_params=pltpu.CompilerParams(needs_layout_passes=False),
    scratch_types=[
        pltpu.VMEM((W,), jnp.float32),
        pltpu.VMEM((W,), jnp.float32),
        pltpu.VMEM((W,), jnp.float32),
        pltpu.VMEM((W,), jnp.float32),
        pltpu.VMEM((HW,), jnp.float32),
        pltpu.VMEM((L,), jnp.float32),
        pltpu.SemaphoreType.DMA,
        pltpu.SemaphoreType.DMA,
        pltpu.SemaphoreType.DMA,
        pltpu.SemaphoreType.DMA,
    ],
)
def _sc_hamming(a_hbm, b_hbm, hist_out, sum_out,
                a0, a1, b0, b1, hist_v, sum_v, sa0, sa1, sb0, sb1):
    wid = lax.axis_index("s") * NC + lax.axis_index("c")
    base = wid * (RPW * D)

    zeros = jnp.zeros((L,), jnp.float32)
    ones = jnp.ones((L,), jnp.float32)
    lane = lax.iota(jnp.int32, L)
    rowsel = lane * D

    for i in range(BINS):
        hist_v[pl.ds(i * L, L)] = zeros
    sum_v[...] = zeros

    def issue(g, aref, bref, sa, sb):
        off = base + g * W
        pltpu.async_copy(a_hbm.at[pl.ds(off, W)], aref, sa)
        pltpu.async_copy(b_hbm.at[pl.ds(off, W)], bref, sb)

    def wait(aref, bref, sa, sb):
        pltpu.make_async_copy(a_hbm.at[pl.ds(0, W)], aref, sa).wait()
        pltpu.make_async_copy(b_hbm.at[pl.ds(0, W)], bref, sb).wait()

    def compute(aref, bref):
        a_t = aref.at[pl.ds(0, W)]
        b_t = bref.at[pl.ds(0, W)]
        hist_t = hist_v.at[pl.ds(0, HW)]
        sum_t = sum_v.at[pl.ds(0, L)]

        def grp(i, carry):
            rbase = i * (L * D) + rowsel
            col = lane
            acc = zeros
            for c in range(D):
                idx = rbase + col
                av = plsc.load_gather(a_t, [idx])
                bv = plsc.load_gather(b_t, [idx])
                acc = acc + jnp.abs(av - bv)
                col = (col + 1) & (D - 1)
            d = acc.astype(jnp.int32)
            plsc.addupdate_scatter(hist_t, [d * L + lane], ones)
            plsc.addupdate(sum_t, acc)
            return carry
        lax.fori_loop(0, GPC, grp, 0)

    issue(0, a0, b0, sa0, sb0)
    issue(1, a1, b1, sa1, sb1)

    def outer(t, carry):
        g = t * 2
        wait(a0, b0, sa0, sb0)
        compute(a0, b0)

        @pl.when(g + 2 < CH)
        def _():
            issue(g + 2, a0, b0, sa0, sb0)

        wait(a1, b1, sa1, sb1)
        compute(a1, b1)

        @pl.when(g + 3 < CH)
        def _():
            issue(g + 3, a1, b1, sa1, sb1)

        return carry

    lax.fori_loop(0, CH // 2, outer, 0)

    pltpu.sync_copy(hist_v, hist_out.at[wid])
    pltpu.sync_copy(sum_v, sum_out.at[wid])


def kernel(y_pred, y_true):
    hist_parts, sum_parts = _sc_hamming(y_pred.reshape(-1), y_true.reshape(-1))
    histogram = hist_parts.reshape(NW, BINS, L).sum(axis=(0, 2))
    mean = sum_parts.sum() / jnp.float32(N)
    return mean, histogram


# speedup vs baseline: 64.4444x; 1.1636x over previous
import functools

import jax
import jax.numpy as jnp
from jax import lax
from jax.experimental import pallas as pl
from jax.experimental.pallas import tpu as pltpu
from jax.experimental.pallas import tpu_sc as plsc

N = 2097152
D = 64
BINS = 65
NC = 2
NS = 16
L = 16
NW = NC * NS
RPW = N // NW
R = 128
CH = RPW // R
W = R * D
GPC = R // L
HW = BINS * L

_mesh = plsc.VectorSubcoreMesh(
    core_axis_name="c", subcore_axis_name="s", num_cores=NC, num_subcores=NS
)


@functools.partial(
    pl.kernel,
    out_type=[
        jax.ShapeDtypeStruct((NW, HW), jnp.float32),
        jax.ShapeDtypeStruct((NW, L), jnp.float32),
    ],
    mesh=_mesh,
    compiler_params=pltpu.CompilerParams(needs_layout_passes=False),
    scratch_types=[
        pltpu.VMEM((R, D), jnp.float32),
        pltpu.VMEM((R, D), jnp.float32),
        pltpu.VMEM((R, D), jnp.float32),
        pltpu.VMEM((R, D), jnp.float32),
        pltpu.VMEM((HW,), jnp.float32),
        pltpu.VMEM((L,), jnp.float32),
        pltpu.SemaphoreType.DMA,
        pltpu.SemaphoreType.DMA,
        pltpu.SemaphoreType.DMA,
        pltpu.SemaphoreType.DMA,
    ],
)
def _sc_hamming(a_hbm, b_hbm, hist_out, sum_out,
                a0, a1, b0, b1, hist_v, sum_v, sa0, sa1, sb0, sb1):
    wid = lax.axis_index("s") * NC + lax.axis_index("c")
    base = wid * RPW

    zeros = jnp.zeros((L,), jnp.float32)
    ones = jnp.ones((L,), jnp.float32)
    lane = lax.iota(jnp.int32, L)

    for i in range(BINS):
        hist_v[pl.ds(i * L, L)] = zeros
    sum_v[...] = zeros

    def issue(g, aref, bref, sa, sb):
        row0 = base + g * R
        pltpu.async_copy(a_hbm.at[pl.ds(row0, R), :], aref, sa)
        pltpu.async_copy(b_hbm.at[pl.ds(row0, R), :], bref, sb)

    def wait(aref, bref, sa, sb):
        pltpu.make_async_copy(a_hbm.at[pl.ds(0, R), :], aref, sa).wait()
        pltpu.make_async_copy(b_hbm.at[pl.ds(0, R), :], bref, sb).wait()

    def compute(aref, bref):
        a_t = aref.at[pl.ds(0, R), :]
        b_t = bref.at[pl.ds(0, R), :]
        hist_t = hist_v.at[pl.ds(0, HW)]
        sum_t = sum_v.at[pl.ds(0, L)]

        def grp(i, carry):
            rows = i * L + lane
            col = lane
            acc = zeros
            for c in range(D):
                av = plsc.load_gather(a_t, [rows, col])
                bv = plsc.load_gather(b_t, [rows, col])
                acc = acc + jnp.abs(av - bv)
                col = (col + 1) & (D - 1)
            d = acc.astype(jnp.int32)
            plsc.addupdate_scatter(hist_t, [d * L + lane], ones)
            plsc.addupdate(sum_t, acc)
            return carry
        lax.fori_loop(0, GPC, grp, 0)

    issue(0, a0, b0, sa0, sb0)
    issue(1, a1, b1, sa1, sb1)

    def outer(t, carry):
        g = t * 2
        wait(a0, b0, sa0, sb0)
        compute(a0, b0)

        @pl.when(g + 2 < CH)
        def _():
            issue(g + 2, a0, b0, sa0, sb0)

        wait(a1, b1, sa1, sb1)
        compute(a1, b1)

        @pl.when(g + 3 < CH)
        def _():
            issue(g + 3, a1, b1, sa1, sb1)

        return carry

    lax.fori_loop(0, CH // 2, outer, 0)

    pltpu.sync_copy(hist_v, hist_out.at[wid])
    pltpu.sync_copy(sum_v, sum_out.at[wid])


def kernel(y_pred, y_true):
    hist_parts, sum_parts = _sc_hamming(y_pred, y_true)
    histogram = hist_parts.reshape(NW, BINS, L).sum(axis=(0, 2))
    mean = sum_parts.sum() / jnp.float32(N)
    return mean, histogram
